# baseline (speedup 1.0000x reference)
import functools

import jax
import jax.numpy as jnp
from jax import lax
from jax.experimental import pallas as pl
from jax.experimental.pallas import tpu as pltpu
from jax.experimental.pallas import tpu_sc as plsc

_B, _T, _V, _P, _D = 4, 8192, 256, 8, 512
_L = 16
_NC, _NS = 2, 16
_NW = _NC * _NS
_TPW = _T // _NW
_C = 64
_NCHUNK = _TPW // _C
_Q = 16
_NQ = _C // _Q
_RPB = _D // _L
_K = _T // _P
_KC = _C // _P
_DP = _P * _D


def _body(ids_hbm, table_hbm, pos_hbm, out_hbm,
          idx_all, pos_v, gat0, gat1, wb0, wb1,
          sg0, sg1, sw0, sw1, sp):
    wid = lax.axis_index("s") * _NC + lax.axis_index("c")
    t_base = wid * _TPW
    k_base = t_base // _P
    gat = [gat0, gat1]
    wb = [wb0, wb1]
    sg = [sg0, sg1]
    sw = [sw0, sw1]

    zero = jnp.zeros((_L,), jnp.float32)

    @pl.when(wid < _B)
    def _pad():
        def zcol(j, carry):
            wb0[0, pl.ds(j * _L, _L)] = zero
            return carry

        lax.fori_loop(0, _DP // _L, zcol, 0)
        pltpu.sync_copy(wb0.at[pl.ds(0, 1)], out_hbm.at[wid, pl.ds(_K, 1)])

    pltpu.sync_copy(ids_hbm.at[:, pl.ds(t_base, _TPW)], idx_all)
    pltpu.async_copy(pos_hbm.at[pl.ds(t_base, _C)], pos_v, sp)
    pltpu.async_copy(table_hbm.at[idx_all.at[0, pl.ds(0, _Q)]], gat0, sg0)

    def wait_write(par):
        pltpu.make_async_copy(wb[par], out_hbm.at[0, pl.ds(0, _KC)], sw[par]).wait()

    def wait_gather(par):
        pltpu.make_async_copy(
            table_hbm.at[idx_all.at[0, pl.ds(0, _Q)]], gat[par], sg[par]).wait()

    def wait_pos():
        pltpu.make_async_copy(pos_hbm.at[pl.ds(0, _C)], pos_v, sp).wait()

    def issue_gather(b, qoff, par):
        pltpu.async_copy(table_hbm.at[idx_all.at[b, qoff]], gat[par], sg[par])

    def chunk_units(c):
        k0 = pl.multiple_of(k_base + c * _KC, _KC)
        for b in range(_B):
            wpar = b & 1
            if b < 2:
                @pl.when(c > 0)
                def _w():
                    wait_write(wpar)
            else:
                wait_write(wpar)
            w = wb[wpar]
            for q in range(_NQ):
                qpar = q & 1
                wait_gather(qpar)
                if q < _NQ - 1:
                    issue_gather(b, pl.ds(c * _C + (q + 1) * _Q, _Q), qpar ^ 1)
                elif b < _B - 1:
                    issue_gather(b + 1, pl.ds(c * _C, _Q), qpar ^ 1)
                else:
                    @pl.when(c + 1 < _NCHUNK)
                    def _g():
                        issue_gather(0, pl.ds((c + 1) * _C, _Q), qpar ^ 1)
                if b == 0 and q == 0:
                    wait_pos()
                g = gat[qpar]

                def row(r, carry):
                    tr = q * _Q + r
                    kr = tr >> 3
                    off = (tr & 7) * _D
                    for j in range(_RPB):
                        w[kr, pl.ds(off + j * _L, _L)] = (
                            g[r, pl.ds(j * _L, _L)] + pos_v[tr, pl.ds(j * _L, _L)])
                    return carry

                lax.fori_loop(0, _Q, row, 0)
            if b == _B - 1:
                @pl.when(c + 1 < _NCHUNK)
                def _p():
                    pltpu.async_copy(
                        pos_hbm.at[pl.ds(t_base + (c + 1) * _C, _C)], pos_v, sp)
            pltpu.async_copy(w, out_hbm.at[b, pl.ds(k0, _KC)], sw[wpar])

    lax.fori_loop(0, _NCHUNK, lambda c, carry: (chunk_units(c), carry)[1], 0)

    wait_write(0)
    wait_write(1)


_kern = functools.partial(
    pl.kernel,
    out_type=jax.ShapeDtypeStruct((_B, _K + 1, _DP), jnp.float32),
    mesh=plsc.VectorSubcoreMesh(core_axis_name="c", subcore_axis_name="s"),
    scratch_types=[
        pltpu.VMEM((_B, _TPW), jnp.int32),
        pltpu.VMEM((_C, _D), jnp.float32),
        pltpu.VMEM((_Q, _D), jnp.float32),
        pltpu.VMEM((_Q, _D), jnp.float32),
        pltpu.VMEM((_KC, _DP), jnp.float32),
        pltpu.VMEM((_KC, _DP), jnp.float32),
        pltpu.SemaphoreType.DMA,
        pltpu.SemaphoreType.DMA,
        pltpu.SemaphoreType.DMA,
        pltpu.SemaphoreType.DMA,
        pltpu.SemaphoreType.DMA,
    ],
)(_body)


@jax.jit
def _megabyte(ids, global_table, pos_table):
    return _kern(ids, global_table, pos_table)


def kernel(ids, global_table, pos_table):
    return _megabyte(ids, global_table, pos_table)


# speedup vs baseline: 2.7067x; 2.7067x over previous
import functools

import jax
import jax.numpy as jnp
from jax import lax
from jax.experimental import pallas as pl
from jax.experimental.pallas import tpu as pltpu
from jax.experimental.pallas import tpu_sc as plsc

_B, _T, _V, _P, _D = 4, 8192, 256, 8, 512
_L = 16
_NC, _NS = 2, 16
_NW = _NC * _NS
_TPW = _T // _NW
_K = _T // _P
_KPW = _TPW // _P
_NPAIR = _KPW // 2
_TPP = 2 * _P
_RPB = _D // _L
_DP = _P * _D


def _body(ids_hbm, table_hbm, pos_hbm, out_hbm,
          idx_all, pos0, pos1, gat0, gat1, wb0, wb1,
          sg0, sg1, sw0, sw1, sp0, sp1):
    wid = lax.axis_index("s") * _NC + lax.axis_index("c")
    t_base = wid * _TPW
    k_base = wid * _KPW
    gat = [gat0, gat1]
    pos = [pos0, pos1]
    wb = [wb0, wb1]
    sg = [sg0, sg1]
    sw = [sw0, sw1]
    sp = [sp0, sp1]

    zero = jnp.zeros((_L,), jnp.float32)

    @pl.when(wid == 0)
    def _pad():
        def zcol(j, carry):
            for b in range(_B):
                wb0[0, b, pl.ds(j * _L, _L)] = zero
            return carry

        lax.fori_loop(0, _DP // _L, zcol, 0)
        pltpu.sync_copy(wb0, out_hbm.at[pl.ds(_K, 1)])

    def issue_pair(j, gpar):
        toff = pl.multiple_of(j * _TPP, _TPP)
        for b in range(_B):
            pltpu.async_copy(table_hbm.at[idx_all.at[b, pl.ds(toff, _TPP)]],
                             gat[gpar].at[pl.ds(b * _TPP, _TPP)], sg[gpar])
        pltpu.async_copy(pos_hbm.at[pl.ds(t_base + toff, _TPP)], pos[gpar],
                         sp[gpar])

    pltpu.sync_copy(ids_hbm.at[:, pl.ds(t_base, _TPW)], idx_all)
    issue_pair(0, 0)
    issue_pair(1, 1)

    def wait_write(par):
        pltpu.make_async_copy(wb[par], out_hbm.at[pl.ds(0, 1)], sw[par]).wait()

    def wait_gather(gpar):
        pltpu.make_async_copy(
            table_hbm.at[idx_all.at[0, pl.ds(0, _B * _TPP)]],
            gat[gpar], sg[gpar]).wait()
        pltpu.make_async_copy(
            pos_hbm.at[pl.ds(0, _TPP)], pos[gpar], sp[gpar]).wait()

    def add_row(gpar, off, wpar):
        g = gat[gpar]
        pv = pos[gpar]
        w = wb[wpar]

        @plsc.parallel_loop(0, _B * _P, step=1, unroll=2)
        def _row(rp):
            b = rp >> 3
            p = rp & (_P - 1)
            gr = b * _TPP + off + p
            pr = off + p
            cb = p * _D
            for j in range(_RPB):
                w[0, b, pl.ds(cb + j * _L, _L)] = (
                    g[gr, pl.ds(j * _L, _L)] + pv[pr, pl.ds(j * _L, _L)])

    def pair_step(j, gpar, first):
        if not first:
            wait_write(0)
        wait_gather(gpar)
        add_row(gpar, 0, 0)
        pltpu.async_copy(wb0, out_hbm.at[pl.ds(k_base + 2 * j, 1)], sw0)
        if not first:
            wait_write(1)
        add_row(gpar, _P, 1)
        pltpu.async_copy(wb1, out_hbm.at[pl.ds(k_base + 2 * j + 1, 1)], sw1)
        @pl.when(j + 2 < _NPAIR)
        def _n():
            issue_pair(j + 2, gpar)

    def quad(qq, carry):
        j0 = 2 * qq
        pair_step(j0, 0, False)
        pair_step(j0 + 1, 1, False)
        return carry

    pair_step(0, 0, True)
    pair_step(1, 1, False)
    lax.fori_loop(1, _NPAIR // 2, quad, 0)

    wait_write(0)
    wait_write(1)


_kern = functools.partial(
    pl.kernel,
    out_type=jax.ShapeDtypeStruct((_K + 1, _B, _DP), jnp.float32),
    mesh=plsc.VectorSubcoreMesh(core_axis_name="c", subcore_axis_name="s"),
    scratch_types=[
        pltpu.VMEM((_B, _TPW), jnp.int32),
        pltpu.VMEM((_TPP, _D), jnp.float32),
        pltpu.VMEM((_TPP, _D), jnp.float32),
        pltpu.VMEM((_B * _TPP, _D), jnp.float32),
        pltpu.VMEM((_B * _TPP, _D), jnp.float32),
        pltpu.VMEM((1, _B, _DP), jnp.float32),
        pltpu.VMEM((1, _B, _DP), jnp.float32),
        pltpu.SemaphoreType.DMA,
        pltpu.SemaphoreType.DMA,
        pltpu.SemaphoreType.DMA,
        pltpu.SemaphoreType.DMA,
        pltpu.SemaphoreType.DMA,
        pltpu.SemaphoreType.DMA,
    ],
)(_body)


@jax.jit
def _megabyte(ids, global_table, pos_table):
    out = _kern(ids, global_table, pos_table)
    return jnp.transpose(out, (1, 0, 2))


def kernel(ids, global_table, pos_table):
    return _megabyte(ids, global_table, pos_table)


# speedup vs baseline: 2.8582x; 1.0560x over previous
import functools

import jax
import jax.numpy as jnp
from jax import lax
from jax.experimental import pallas as pl
from jax.experimental.pallas import tpu as pltpu
from jax.experimental.pallas import tpu_sc as plsc

_B, _T, _V, _P, _D = 4, 8192, 256, 8, 512
_L = 16
_NC, _NS = 2, 16
_NW = _NC * _NS
_TPW = _T // _NW
_K = _T // _P
_KPW = _TPW // _P
_CP = 64
_UPC = _CP // _P
_RPB = _D // _L
_DP = _P * _D


def _body(ids_hbm, table_hbm, pos_hbm, out_hbm,
          idx_all, pos_v, gat0, gat1, wb0, wb1,
          sg0, sg1, sw0, sw1, sp):
    wid = lax.axis_index("s") * _NC + lax.axis_index("c")
    t_base = wid * _TPW
    k_base = wid * _KPW
    gat = [gat0, gat1]
    wb = [wb0, wb1]
    sg = [sg0, sg1]
    sw = [sw0, sw1]

    zero = jnp.zeros((_L,), jnp.float32)

    @pl.when(wid == 0)
    def _pad():
        def zcol(j, carry):
            for b in range(_B):
                wb0[0, b, pl.ds(j * _L, _L)] = zero
            return carry

        lax.fori_loop(0, _DP // _L, zcol, 0)
        pltpu.sync_copy(wb0, out_hbm.at[pl.ds(_K, 1)])

    pltpu.sync_copy(ids_hbm.at[:, pl.ds(t_base, _TPW)], idx_all)
    pltpu.async_copy(pos_hbm.at[pl.ds(t_base, _CP)], pos_v, sp)
    for b in range(_B):
        pltpu.async_copy(table_hbm.at[idx_all.at[b, pl.ds(0, _P)]],
                         gat0.at[pl.ds(b * _P, _P)], sg0)

    def wait_write(par):
        pltpu.make_async_copy(wb[par], out_hbm.at[pl.ds(0, 1)], sw[par]).wait()

    def wait_gather(par):
        pltpu.make_async_copy(
            table_hbm.at[idx_all.at[0, pl.ds(0, _B * _P)]], gat[par], sg[par]).wait()

    def wait_pos():
        pltpu.make_async_copy(pos_hbm.at[pl.ds(0, _CP)], pos_v, sp).wait()

    def unit(u, par):
        npar = par ^ 1
        @pl.when(u + 1 < _KPW)
        def _g():
            toff = pl.multiple_of((u + 1) * _P, _P)
            for b in range(_B):
                pltpu.async_copy(table_hbm.at[idx_all.at[b, pl.ds(toff, _P)]],
                                 gat[npar].at[pl.ds(b * _P, _P)], sg[npar])

        @pl.when((u & (_UPC - 1)) == 0)
        def _wp():
            wait_pos()

        @pl.when(u >= 2)
        def _ww():
            wait_write(par)

        wait_gather(par)

        g = gat[par]
        w = wb[par]
        prow = (u & (_UPC - 1)) * _P

        @plsc.parallel_loop(0, _B * _P, step=1, unroll=2)
        def _row(rp):
            b = rp >> 3
            p = rp & (_P - 1)
            pr = prow + p
            cb = p * _D
            for j in range(_RPB):
                w[0, b, pl.ds(cb + j * _L, _L)] = (
                    g[rp, pl.ds(j * _L, _L)] + pos_v[pr, pl.ds(j * _L, _L)])

        @pl.when(((u & (_UPC - 1)) == (_UPC - 1)) & (u + 1 < _KPW))
        def _pp():
            pltpu.async_copy(
                pos_hbm.at[pl.ds(t_base + (u + 1) * _P, _CP)], pos_v, sp)

        pltpu.async_copy(w, out_hbm.at[pl.ds(k_base + u, 1)], sw[par])

    def pair(uu, carry):
        unit(2 * uu, 0)
        unit(2 * uu + 1, 1)
        return carry

    lax.fori_loop(0, _KPW // 2, pair, 0)

    wait_write(0)
    wait_write(1)


_kern = functools.partial(
    pl.kernel,
    out_type=jax.ShapeDtypeStruct((_K + 1, _B, _DP), jnp.float32),
    mesh=plsc.VectorSubcoreMesh(core_axis_name="c", subcore_axis_name="s"),
    scratch_types=[
        pltpu.VMEM((_B, _TPW), jnp.int32),
        pltpu.VMEM((_CP, _D), jnp.float32),
        pltpu.VMEM((_B * _P, _D), jnp.float32),
        pltpu.VMEM((_B * _P, _D), jnp.float32),
        pltpu.VMEM((1, _B, _DP), jnp.float32),
        pltpu.VMEM((1, _B, _DP), jnp.float32),
        pltpu.SemaphoreType.DMA,
        pltpu.SemaphoreType.DMA,
        pltpu.SemaphoreType.DMA,
        pltpu.SemaphoreType.DMA,
        pltpu.SemaphoreType.DMA,
    ],
)(_body)


@jax.jit
def _megabyte(ids, global_table, pos_table):
    out = _kern(ids, global_table, pos_table)
    return jnp.transpose(out, (1, 0, 2))


def kernel(ids, global_table, pos_table):
    return _megabyte(ids, global_table, pos_table)
